# Initial kernel scaffold; baseline (speedup 1.0000x reference)
import jax
import jax.numpy as jnp
from jax.experimental import pallas as pl


def kernel(t, n_tokens, table):
    raise NotImplementedError("write your pallas kernel here")


# speedup vs baseline: 1.0199x; 1.0199x over previous
import functools

import jax
import jax.numpy as jnp
from jax.experimental import pallas as pl
from jax.experimental.pallas import tpu as pltpu

B = 4096
T = 200
D = 128
VPAD = 64

BB = 32
GRID = B // BB


def _tc_body(t_ref, table_ref, out_ref):
    idx = t_ref[0, 0, :]
    onehot = (idx[:, None] == jax.lax.broadcasted_iota(jnp.int32, (BB, VPAD), 1)
              ).astype(jnp.float32)
    emb = jnp.dot(onehot, table_ref[...], preferred_element_type=jnp.float32)
    out_ref[...] = jnp.broadcast_to(emb[:, None, :], (BB, T, D))


@jax.jit
def _run(t, table):
    t3 = t.reshape(GRID, 1, BB).astype(jnp.int32)
    table_p = jnp.pad(table, ((0, VPAD - table.shape[0]), (0, 0)))
    return pl.pallas_call(
        _tc_body,
        grid=(GRID,),
        in_specs=[
            pl.BlockSpec((1, 1, BB), lambda i: (i, 0, 0)),
            pl.BlockSpec((VPAD, D), lambda i: (0, 0)),
        ],
        out_specs=pl.BlockSpec((BB, T, D), lambda i: (i, 0, 0)),
        out_shape=jax.ShapeDtypeStruct((B, T, D), jnp.float32),
    )(t3, table_p)


def kernel(t, n_tokens, table):
    del n_tokens
    return _run(t, table)


# speedup vs baseline: 1.1047x; 1.0832x over previous
import functools

import jax
import jax.numpy as jnp
from jax.experimental import pallas as pl
from jax.experimental.pallas import tpu as pltpu

B = 4096
T = 200
D = 128
V = 60

BB = 64
GRID = B // BB


def _tc_body(t_ref, table_ref, out_ref):
    idx = t_ref[0, 0, :]
    onehot = (idx[:, None] == jax.lax.broadcasted_iota(jnp.int32, (BB, V), 1)
              ).astype(jnp.float32)
    emb = jnp.dot(onehot, table_ref[...], preferred_element_type=jnp.float32)
    out_ref[...] = jnp.broadcast_to(emb[:, None, :], (BB, T, D))


@jax.jit
def _run(t, table):
    t3 = t.reshape(GRID, 1, BB)
    return pl.pallas_call(
        _tc_body,
        grid=(GRID,),
        in_specs=[
            pl.BlockSpec((1, 1, BB), lambda i: (i, 0, 0)),
            pl.BlockSpec((V, D), lambda i: (0, 0)),
        ],
        out_specs=pl.BlockSpec((BB, T, D), lambda i: (i, 0, 0)),
        out_shape=jax.ShapeDtypeStruct((B, T, D), jnp.float32),
    )(t3, table)


def kernel(t, n_tokens, table):
    del n_tokens
    return _run(t, table)
